# Initial kernel scaffold; baseline (speedup 1.0000x reference)
import jax
import jax.numpy as jnp
from jax.experimental import pallas as pl


def kernel(input_, weight):
    raise NotImplementedError("write your pallas kernel here")


# speedup vs baseline: 1.8772x; 1.8772x over previous
import jax
import jax.numpy as jnp
from jax import lax
from jax.experimental import pallas as pl
from jax.experimental.pallas import tpu as pltpu
from jax.experimental.pallas import tpu_sc as plsc

NUM_EMBEDDINGS = 1000000
EMBEDDING_DIM = 64
BATCH = 16384
HIST = 50

_INFO = plsc.get_sparse_core_info()
NC = _INFO.num_cores
NS = _INFO.num_subcores
NW = NC * NS

B_TOTAL = BATCH * HIST
B_PER_W = B_TOTAL // NW
CHUNK = 128
N_CHUNKS = B_PER_W // CHUNK
NBUF = 4
N_LOOP = N_CHUNKS // NBUF

assert B_PER_W * NW == B_TOTAL
assert CHUNK * N_CHUNKS == B_PER_W
assert NBUF * N_LOOP == N_CHUNKS


def _body(table_hbm, idx_hbm, out_hbm, idx_v, *bufs):
  rows = bufs[:NBUF]
  sems = bufs[NBUF:]
  wid = lax.axis_index("s") * NC + lax.axis_index("c")
  row_base = wid * B_PER_W

  pltpu.sync_copy(idx_hbm.at[pl.ds(wid * N_CHUNKS, N_CHUNKS)], idx_v)

  def start_gather(g, b):
    pltpu.async_copy(table_hbm.at[idx_v.at[g]], rows[b], sems[b])

  def wait_gather(g, b):
    pltpu.make_async_copy(table_hbm.at[idx_v.at[g]], rows[b], sems[b]).wait()

  for b in range(NBUF):
    start_gather(b, b)

  @pl.loop(0, N_LOOP)
  def _(t):
    for b in range(NBUF):
      g = t * NBUF + b
      wait_gather(g, b)
      pltpu.sync_copy(rows[b], out_hbm.at[pl.ds(row_base + g * CHUNK, CHUNK)])

      @pl.when(t < N_LOOP - 1)
      def _():
        start_gather(g + NBUF, b)


@jax.jit
def kernel(input_, weight):
  idx = input_.astype(jnp.int32).reshape(NW * N_CHUNKS, CHUNK)

  mesh = plsc.VectorSubcoreMesh(core_axis_name="c", subcore_axis_name="s")
  gathered = pl.kernel(
      _body,
      out_type=jax.ShapeDtypeStruct((B_TOTAL, EMBEDDING_DIM), jnp.float32),
      mesh=mesh,
      compiler_params=pltpu.CompilerParams(use_tc_tiling_on_sc=False),
      scratch_types=(
          [pltpu.VMEM((N_CHUNKS, CHUNK), jnp.int32)]
          + [pltpu.VMEM((CHUNK, EMBEDDING_DIM), jnp.float32) for _ in range(NBUF)]
          + [pltpu.SemaphoreType.DMA for _ in range(NBUF)]
      ),
  )(weight, idx)

  return gathered.reshape(BATCH, HIST, EMBEDDING_DIM)
